# Initial kernel scaffold; baseline (speedup 1.0000x reference)
import jax
import jax.numpy as jnp
from jax.experimental import pallas as pl


def kernel(p0, p1, p2, targets, mapped_anchors):
    raise NotImplementedError("write your pallas kernel here")


# speedup vs baseline: 1.5523x; 1.5523x over previous
import functools
import math

import jax
import jax.numpy as jnp
from jax import lax
from jax.experimental import pallas as pl
from jax.experimental.pallas import tpu as pltpu

NCLS = 80
NANCH = 3
NLVL = 3
BAL = (4.0, 1.0, 0.4)
GBIAS = 0.5
EPAD = 4608

_FEAT = ((80, 80), (40, 40), (20, 20))

_ATAN_C = (0.9999999937488345, -0.33333137929908097, 0.19993693394198278,
           -0.14211098330283195, 0.10667454712913349, -0.07556827050084194,
           0.04327732083475509, -0.01641258775269415, 0.002932602096126738)


def _atan_pos(z):
    inv = z > 1.0
    x = jnp.where(inv, 1.0 / z, z)
    x2 = x * x
    q = jnp.full_like(x, _ATAN_C[-1])
    for c in _ATAN_C[-2::-1]:
        q = q * x2 + c
    a = x * q
    return jnp.where(inv, math.pi / 2 - a, a)
_M = tuple(16 * 3 * h * w for (w, h) in _FEAT)
_RB = 960
_SEG = tuple(m // _RB for m in _M)


def _prep_targets(targets, mapped_anchors):
    nt = targets.shape[0]
    ai = jnp.tile(jnp.arange(NANCH, dtype=jnp.float32).reshape(NANCH, 1), (1, nt))
    t_all = jnp.concatenate(
        [jnp.tile(targets[None], (NANCH, 1, 1)), ai[..., None]], axis=-1)
    off = jnp.array([[0, 0], [1, 0], [0, 1], [-1, 0], [0, -1]],
                    dtype=jnp.float32) * GBIAS
    out = []
    for i in range(NLVL):
        anchors = mapped_anchors[i]
        gw, gh = _FEAT[i]
        gain = jnp.array([1.0, 1.0, gw, gh, gw, gh, 1.0], dtype=jnp.float32)
        t = t_all * gain
        r = t[..., 4:6] / anchors[:, None]
        fmask = jnp.max(jnp.maximum(r, 1.0 / r), axis=2) < 4.0
        t = t.reshape(NANCH * nt, 7)
        vmask = fmask.reshape(NANCH * nt)
        gxy = t[:, 2:4]
        gxi = jnp.array([gw, gh], dtype=jnp.float32) - gxy
        jk = (gxy % 1 < GBIAS) & (gxy > 1)
        lm = (gxi % 1 < GBIAS) & (gxi > 1)
        jm, km = jk[:, 0], jk[:, 1]
        lmm, mm = lm[:, 0], lm[:, 1]
        jmask = jnp.stack([jnp.ones_like(jm), jm, km, lmm, mm])
        mask = (jmask & vmask[None]).reshape(5 * NANCH * nt)
        t = jnp.tile(t[None], (5, 1, 1)).reshape(5 * NANCH * nt, 7)
        offsets = (jnp.zeros_like(gxy)[None] + off[:, None]).reshape(
            5 * NANCH * nt, 2)
        bc = t[:, :2]
        gxy2 = t[:, 2:4]
        gwh = t[:, 4:6]
        aidx = t[:, 6].astype(jnp.int32)
        b = bc[:, 0].astype(jnp.int32)
        cls = bc[:, 1].astype(jnp.int32)
        gij = (gxy2 - offsets).astype(jnp.int32)
        gi = jnp.clip(gij[:, 0], 0, gw - 1)
        gj = jnp.clip(gij[:, 1], 0, gh - 1)
        bbox = jnp.concatenate(
            [gxy2 - jnp.stack([gi, gj], axis=1).astype(jnp.float32), gwh], axis=1)
        anchor = anchors[aidx]
        idx = ((b * NANCH + aidx) * gh + gj) * gw + gi
        idx = jnp.clip(idx, 0, _M[i] - 1)
        out.append((idx, bbox, anchor, cls, mask))
    return out


def _pad_to(x, n, val=0):
    return jnp.pad(x, [(0, n - x.shape[0])] + [(0, 0)] * (x.ndim - 1),
                   constant_values=val)


def _main_body(p0_ref, p1_ref, p2_ref, planes_ref, cols_ref, maskc_ref,
               clsc_ref, g0_ref, g1_ref, g2_ref, o_ref, acc_ref):
    i = pl.program_id(0)

    @pl.when(i == 0)
    def _init():
        for k in range(NLVL):
            acc_ref[k] = 0.0

    bounds = []
    s = 0
    for k in range(NLVL):
        bounds.append((s, s + _SEG[k]))
        s += _SEG[k]

    for k, ref in enumerate((p0_ref, p1_ref, p2_ref)):
        lo, hi = bounds[k]

        @pl.when((i >= lo) & (i < hi))
        def _dense(ref=ref, k=k):
            x = ref[:, 4:5]
            sp = jnp.maximum(x, 0.0) + jnp.log1p(jnp.exp(-jnp.abs(x)))
            acc_ref[k] += jnp.sum(sp)

    @pl.when(i == s - 1)
    def _finalize():
        box_tot = 0.0
        conf_tot = 0.0
        cls_tot = 0.0
        eps = 1e-07
        for l, g_ref in enumerate((g0_ref, g1_ref, g2_ref)):
            bx = planes_ref[l, 0]
            by = planes_ref[l, 1]
            bw = planes_ref[l, 2]
            bh = planes_ref[l, 3]
            aw = planes_ref[l, 4]
            ah = planes_ref[l, 5]
            mk = planes_ref[l, 6]
            px = cols_ref[l, 0]
            py = cols_ref[l, 1]
            pw = cols_ref[l, 2]
            ph = cols_ref[l, 3]
            x4 = cols_ref[l, 4]
            sig = lambda z: 1.0 / (1.0 + jnp.exp(-z))
            pxv = sig(px) * 2.0 - 0.5
            pyv = sig(py) * 2.0 - 0.5
            pwv = (sig(pw) * 2.0) ** 2 * aw
            phv = (sig(ph) * 2.0) ** 2 * ah
            b1x1, b1x2 = pxv - pwv / 2, pxv + pwv / 2
            b1y1, b1y2 = pyv - phv / 2, pyv + phv / 2
            b2x1, b2x2 = bx - bw / 2, bx + bw / 2
            b2y1, b2y2 = by - bh / 2, by + bh / 2
            inter = (jnp.clip(jnp.minimum(b1x2, b2x2) - jnp.maximum(b1x1, b2x1),
                              0.0, None)
                     * jnp.clip(jnp.minimum(b1y2, b2y2) - jnp.maximum(b1y1, b2y1),
                                0.0, None))
            union = pwv * phv + bw * bh - inter + eps
            iou = inter / union
            cw = jnp.maximum(b1x2, b2x2) - jnp.minimum(b1x1, b2x1)
            ch = jnp.maximum(b1y2, b2y2) - jnp.minimum(b1y1, b2y1)
            c2 = cw ** 2 + ch ** 2 + eps
            rho2 = ((b2x1 + b2x2 - b1x1 - b1x2) ** 2
                    + (b2y1 + b2y2 - b1y1 - b1y2) ** 2) / 4.0
            v = 4.0 / math.pi ** 2 * (_atan_pos(bw / bh)
                                      - _atan_pos(pwv / phv)) ** 2
            alpha = v / (v - iou + (1.0 + eps))
            iou_c = iou - (rho2 / c2 + v * alpha)

            denom = jnp.maximum(jnp.sum(mk), 1.0)
            box_tot += jnp.sum((1.0 - iou_c) * mk) / denom
            iou_d = jnp.clip(iou_c, 0.0, None)
            sub = jnp.sum(mk * x4 * iou_d)
            conf_tot += (acc_ref[l] - sub) / float(_M[l]) * BAL[l]

            pcls = g_ref[:, 5:5 + NCLS]
            mc = maskc_ref[l]
            cc = clsc_ref[l]
            onehot = (lax.broadcasted_iota(jnp.int32, (EPAD, NCLS), 1)
                      .astype(jnp.float32) == cc).astype(jnp.float32)
            closs = (jnp.maximum(pcls, 0.0) - pcls * onehot
                     + jnp.log1p(jnp.exp(-jnp.abs(pcls))))
            cls_tot += jnp.sum(closs * mc) / (denom * NCLS)

        total = (box_tot * 0.05 + conf_tot * 1.0 + cls_tot * 0.5) * 16.0
        o_ref[...] = jnp.broadcast_to(total, (1, 1))


def _run_main(p0r, p1r, p2r, planes, cols, maskc, clsc, g0, g1, g2):
    nsteps = sum(_SEG)
    b0, b1 = _SEG[0], _SEG[0] + _SEG[1]
    in_specs = [
            pl.BlockSpec((_RB, 85), lambda i: (jnp.minimum(i, b0 - 1), 0)),
            pl.BlockSpec((_RB, 85),
                         lambda i: (jnp.clip(i - b0, 0, _SEG[1] - 1), 0)),
            pl.BlockSpec((_RB, 85),
                         lambda i: (jnp.clip(i - b1, 0, _SEG[2] - 1), 0)),
            pl.BlockSpec((NLVL, 8, 36, 128), lambda i: (0, 0, 0, 0)),
            pl.BlockSpec((NLVL, 5, 36, 128), lambda i: (0, 0, 0, 0)),
            pl.BlockSpec((NLVL, EPAD, 1), lambda i: (0, 0, 0)),
            pl.BlockSpec((NLVL, EPAD, 1), lambda i: (0, 0, 0)),
            pl.BlockSpec((EPAD, 85), lambda i: (0, 0)),
            pl.BlockSpec((EPAD, 85), lambda i: (0, 0)),
            pl.BlockSpec((EPAD, 85), lambda i: (0, 0)),
        ]
    return pl.pallas_call(
        _main_body,
        grid=(nsteps,),
        in_specs=in_specs,
        out_specs=pl.BlockSpec((1, 1), lambda i: (0, 0)),
        out_shape=jax.ShapeDtypeStruct((1, 1), jnp.float32),
        scratch_shapes=[pltpu.SMEM((NLVL,), jnp.float32)],
    )(p0r, p1r, p2r, planes, cols, maskc, clsc, g0, g1, g2)


def kernel(p0, p1, p2, targets, mapped_anchors):
    prs = [p.reshape(-1, 85) for p in (p0, p1, p2)]
    prep = _prep_targets(targets, mapped_anchors)

    planes_l, cols_l, maskc_l, clsc_l, gs = [], [], [], [], []
    for i in range(NLVL):
        idx, bbox, anchor, cls, mask = prep[i]
        idxp = _pad_to(idx, EPAD)
        maskf = _pad_to(mask.astype(jnp.float32), EPAD)
        clsf = _pad_to(cls.astype(jnp.float32), EPAD)
        bboxp = _pad_to(bbox, EPAD, 1.0)
        anchp = _pad_to(anchor, EPAD, 1.0)
        g = prs[i][idxp]
        gs.append(g)
        planes = jnp.stack([
            bboxp[:, 0], bboxp[:, 1], bboxp[:, 2], bboxp[:, 3],
            anchp[:, 0], anchp[:, 1], maskf, clsf,
        ]).reshape(8, 36, 128)
        planes_l.append(planes)
        cols_l.append(g[:, :5].T.reshape(5, 36, 128))
        maskc_l.append(maskf.reshape(EPAD, 1))
        clsc_l.append(clsf.reshape(EPAD, 1))

    out = _run_main(prs[0], prs[1], prs[2],
                    jnp.stack(planes_l), jnp.stack(cols_l),
                    jnp.stack(maskc_l), jnp.stack(clsc_l), *gs)
    return out.reshape(1)
